# baseline (speedup 1.0000x reference)
import functools

import jax
import jax.numpy as jnp
from jax import lax
from jax.experimental import pallas as pl
from jax.experimental.pallas import tpu as pltpu
from jax.experimental.pallas import tpu_sc as plsc

NE = 8
D = 1024
F = 1024
BS = 4096
HBS = 2048
HNA = 2 * HBS
G = 256
HNB = (HNA + NE * (G - 1) + G - 1) // G
HPAD = HNB * G

_LANES = 128

NW = 32
SC_CH = 64


def _route_body(x_ref, wr_ref, idx_ref, w0_ref, w1_ref, be_ref, rank_ref):
    x = x_ref[...]
    wr = wr_ref[...]
    logits = jnp.dot(x, wr, preferred_element_type=jnp.float32)
    lane = lax.broadcasted_iota(jnp.int32, (HBS, _LANES), 1)
    valid = lane < NE
    neg = jnp.float32(-1e30)
    ml = jnp.where(valid, logits, neg)

    m1 = jnp.max(ml, axis=1, keepdims=True)
    idx1 = jnp.min(jnp.where(ml == m1, lane, _LANES), axis=1, keepdims=True)
    oh0 = (lane == idx1).astype(jnp.float32)
    ml2 = jnp.where(lane == idx1, neg, ml)
    m2 = jnp.max(ml2, axis=1, keepdims=True)
    idx2 = jnp.min(jnp.where(ml2 == m2, lane, _LANES), axis=1, keepdims=True)
    oh1 = (lane == idx2).astype(jnp.float32)

    w0_ref[...] = jax.nn.sigmoid(m1 - m2)
    w1_ref[...] = jax.nn.sigmoid(m2 - m1)

    rank_ref[...] = oh0 + oh1
    tri = (lax.broadcasted_iota(jnp.int32, (128, 128), 0) >
           lax.broadcasted_iota(jnp.int32, (128, 128), 1)).astype(jnp.float32)

    def chunk(c, carry):
        ch = rank_ref[pl.ds(c * 128, 128), :]
        within = jnp.dot(tri, ch, preferred_element_type=jnp.float32)
        rank_ref[pl.ds(c * 128, 128), :] = within + carry
        return carry + jnp.sum(ch, axis=0, keepdims=True)

    counts = lax.fori_loop(0, HBS // 128, chunk,
                           jnp.zeros((1, _LANES), jnp.float32))
    rank = rank_ref[...]

    pc = jnp.floor((counts + (G - 1)) * (1.0 / G)) * G
    upper = (lax.broadcasted_iota(jnp.int32, (128, 128), 0) <
             lax.broadcasted_iota(jnp.int32, (128, 128), 1)).astype(jnp.float32)
    seg = jnp.dot(pc, upper, preferred_element_type=jnp.float32)

    pos = seg + rank
    idx_ref[0:HBS, :] = jnp.sum(pos * oh0, axis=1,
                                keepdims=True).astype(jnp.int32)
    idx_ref[HBS:HNA, :] = jnp.sum(pos * oh1, axis=1,
                                  keepdims=True).astype(jnp.int32)

    bstart = (lax.broadcasted_iota(jnp.int32, (HNB, _LANES), 0) * G
              ).astype(jnp.float32)
    lane_b = lax.broadcasted_iota(jnp.int32, (HNB, _LANES), 1)
    ind = (bstart >= seg) & (bstart < seg + pc)
    be_ref[...] = jnp.sum(
        jnp.where(ind, lane_b, 0), axis=1, keepdims=True).astype(jnp.int32)


def _route_plan(flat_h, wr_pad):
    return pl.pallas_call(
        _route_body,
        out_shape=(
            jax.ShapeDtypeStruct((HNA, 1), jnp.int32),
            jax.ShapeDtypeStruct((HBS, 1), jnp.float32),
            jax.ShapeDtypeStruct((HBS, 1), jnp.float32),
            jax.ShapeDtypeStruct((HNB, 1), jnp.int32),
        ),
        scratch_shapes=[pltpu.VMEM((HBS, _LANES), jnp.float32)],
        compiler_params=pltpu.CompilerParams(
            vmem_limit_bytes=64 * 1024 * 1024),
    )(flat_h, wr_pad)


def _sc_scatter_rows(flat_h, idx):
    bpw = HNA // NW

    @functools.partial(
        pl.kernel,
        mesh=plsc.VectorSubcoreMesh(core_axis_name="c", subcore_axis_name="s"),
        out_type=jax.ShapeDtypeStruct((HPAD, D), jnp.float32),
        scratch_types=[
            pltpu.VMEM((SC_CH,), jnp.int32),
            pltpu.VMEM((SC_CH, D), jnp.float32),
            pltpu.SemaphoreType.DMA,
        ],
    )
    def k(x_hbm, idx_hbm, xs_hbm, idx_v, rows_v, sem):
        wid = lax.axis_index("s") * 2 + lax.axis_index("c")
        base = wid * bpw

        @pl.loop(0, bpw // SC_CH)
        def _(ci):
            off = base + ci * SC_CH
            xoff = lax.rem(off, HBS)
            pltpu.sync_copy(idx_hbm.at[pl.ds(off, SC_CH)], idx_v)
            pltpu.sync_copy(x_hbm.at[pl.ds(xoff, SC_CH)], rows_v)
            pltpu.async_copy(rows_v, xs_hbm.at[idx_v], sem).wait()

    return k(flat_h, idx)


def _grouped_body(be_ref, xs_ref, wg_ref, wu_ref, wd_ref, ys_ref):
    xb = xs_ref[...]
    g = jnp.dot(xb, wg_ref[0], preferred_element_type=jnp.float32)
    u = jnp.dot(xb, wu_ref[0], preferred_element_type=jnp.float32)
    h = g * jax.nn.sigmoid(g) * u
    ys_ref[...] = jnp.dot(h, wd_ref[0], preferred_element_type=jnp.float32)


def _grouped_mlp(be, xs, wg, wu, wd):
    grid_spec = pltpu.PrefetchScalarGridSpec(
        num_scalar_prefetch=1,
        grid=(HNB,),
        in_specs=[
            pl.BlockSpec((G, D), lambda i, be: (i, 0)),
            pl.BlockSpec((1, D, F), lambda i, be: (be[i], 0, 0)),
            pl.BlockSpec((1, D, F), lambda i, be: (be[i], 0, 0)),
            pl.BlockSpec((1, F, D), lambda i, be: (be[i], 0, 0)),
        ],
        out_specs=pl.BlockSpec((G, D), lambda i, be: (i, 0)),
    )
    return pl.pallas_call(
        _grouped_body,
        grid_spec=grid_spec,
        out_shape=jax.ShapeDtypeStruct((HPAD, D), jnp.float32),
        compiler_params=pltpu.CompilerParams(
            dimension_semantics=("arbitrary",)),
    )(be, xs, wg, wu, wd)


def _sc_gather_rows(ys, idx):
    bpw = HNA // NW

    @functools.partial(
        pl.kernel,
        mesh=plsc.VectorSubcoreMesh(core_axis_name="c", subcore_axis_name="s"),
        out_type=jax.ShapeDtypeStruct((HNA, D), jnp.float32),
        scratch_types=[
            pltpu.VMEM((SC_CH,), jnp.int32),
            pltpu.VMEM((SC_CH, D), jnp.float32),
            pltpu.SemaphoreType.DMA,
        ],
    )
    def k(ys_hbm, idx_hbm, g_hbm, idx_v, rows_v, sem):
        wid = lax.axis_index("s") * 2 + lax.axis_index("c")
        base = wid * bpw

        @pl.loop(0, bpw // SC_CH)
        def _(ci):
            off = base + ci * SC_CH
            pltpu.sync_copy(idx_hbm.at[pl.ds(off, SC_CH)], idx_v)
            pltpu.async_copy(ys_hbm.at[idx_v], rows_v, sem).wait()
            pltpu.sync_copy(rows_v, g_hbm.at[pl.ds(off, SC_CH)])

    return k(ys, idx)


def _combine_body(g0_ref, g1_ref, w0_ref, w1_ref, out_ref):
    out_ref[...] = w0_ref[...] * g0_ref[...] + w1_ref[...] * g1_ref[...]


_RB = 512


def _combine(g, w0, w1):
    nblk = HBS // _RB
    return pl.pallas_call(
        _combine_body,
        grid=(nblk,),
        in_specs=[
            pl.BlockSpec((_RB, D), lambda i: (i, 0)),
            pl.BlockSpec((_RB, D), lambda i: (i + nblk, 0)),
            pl.BlockSpec((_RB, 1), lambda i: (i, 0)),
            pl.BlockSpec((_RB, 1), lambda i: (i, 0)),
        ],
        out_specs=pl.BlockSpec((_RB, D), lambda i: (i, 0)),
        out_shape=jax.ShapeDtypeStruct((HBS, D), jnp.float32),
    )(g, g, w0, w1)


def kernel(hidden_states, W_router, W_gate, W_up, W_down):
    B, S, H = hidden_states.shape
    flat = hidden_states.reshape(BS, D)
    wr_pad = jnp.pad(W_router, ((0, 0), (0, _LANES - NE)))

    plans = []
    for h in range(2):
        flat_h = lax.slice(flat, (h * HBS, 0), ((h + 1) * HBS, D))
        idx, w0, w1, be = _route_plan(flat_h, wr_pad)
        plans.append((flat_h, idx.reshape(HNA), w0, w1, be.reshape(HNB)))

    halves = []
    for h in range(2):
        flat_h, idx, w0, w1, be = plans[h]
        xs = _sc_scatter_rows(flat_h, idx)
        ys = _grouped_mlp(be, xs, W_gate, W_up, W_down)
        g = _sc_gather_rows(ys, idx)
        halves.append(_combine(g, w0, w1))

    out = jnp.concatenate(halves, axis=0)
    return out.reshape(B, S, H)


# speedup vs baseline: 1.0065x; 1.0065x over previous
import functools

import jax
import jax.numpy as jnp
from jax import lax
from jax.experimental import pallas as pl
from jax.experimental.pallas import tpu as pltpu
from jax.experimental.pallas import tpu_sc as plsc

NE = 8
D = 1024
F = 1024
BS = 4096
HBS = 2048
HNA = 2 * HBS
G = 512
HNB = (HNA + NE * (G - 1) + G - 1) // G
HPAD = HNB * G

_LANES = 128

NW = 32
SC_CH = 64


def _route_body(x_ref, wr_ref, idx_ref, w0_ref, w1_ref, be_ref, rank_ref):
    x = x_ref[...]
    wr = wr_ref[...]
    logits = jnp.dot(x, wr, preferred_element_type=jnp.float32)
    lane = lax.broadcasted_iota(jnp.int32, (HBS, _LANES), 1)
    valid = lane < NE
    neg = jnp.float32(-1e30)
    ml = jnp.where(valid, logits, neg)

    m1 = jnp.max(ml, axis=1, keepdims=True)
    idx1 = jnp.min(jnp.where(ml == m1, lane, _LANES), axis=1, keepdims=True)
    oh0 = (lane == idx1).astype(jnp.float32)
    ml2 = jnp.where(lane == idx1, neg, ml)
    m2 = jnp.max(ml2, axis=1, keepdims=True)
    idx2 = jnp.min(jnp.where(ml2 == m2, lane, _LANES), axis=1, keepdims=True)
    oh1 = (lane == idx2).astype(jnp.float32)

    w0_ref[...] = jax.nn.sigmoid(m1 - m2)
    w1_ref[...] = jax.nn.sigmoid(m2 - m1)

    rank_ref[...] = oh0 + oh1
    tri = (lax.broadcasted_iota(jnp.int32, (128, 128), 0) >
           lax.broadcasted_iota(jnp.int32, (128, 128), 1)).astype(jnp.float32)

    def chunk(c, carry):
        ch = rank_ref[pl.ds(c * 128, 128), :]
        within = jnp.dot(tri, ch, preferred_element_type=jnp.float32)
        rank_ref[pl.ds(c * 128, 128), :] = within + carry
        return carry + jnp.sum(ch, axis=0, keepdims=True)

    counts = lax.fori_loop(0, HBS // 128, chunk,
                           jnp.zeros((1, _LANES), jnp.float32))
    rank = rank_ref[...]

    pc = jnp.floor((counts + (G - 1)) * (1.0 / G)) * G
    upper = (lax.broadcasted_iota(jnp.int32, (128, 128), 0) <
             lax.broadcasted_iota(jnp.int32, (128, 128), 1)).astype(jnp.float32)
    seg = jnp.dot(pc, upper, preferred_element_type=jnp.float32)

    pos = seg + rank
    idx_ref[0:HBS, :] = jnp.sum(pos * oh0, axis=1,
                                keepdims=True).astype(jnp.int32)
    idx_ref[HBS:HNA, :] = jnp.sum(pos * oh1, axis=1,
                                  keepdims=True).astype(jnp.int32)

    bstart = (lax.broadcasted_iota(jnp.int32, (HNB, _LANES), 0) * G
              ).astype(jnp.float32)
    lane_b = lax.broadcasted_iota(jnp.int32, (HNB, _LANES), 1)
    ind = (bstart >= seg) & (bstart < seg + pc)
    be_ref[...] = jnp.sum(
        jnp.where(ind, lane_b, 0), axis=1, keepdims=True).astype(jnp.int32)


def _route_plan(flat_h, wr_pad):
    return pl.pallas_call(
        _route_body,
        out_shape=(
            jax.ShapeDtypeStruct((HNA, 1), jnp.int32),
            jax.ShapeDtypeStruct((HBS, 1), jnp.float32),
            jax.ShapeDtypeStruct((HBS, 1), jnp.float32),
            jax.ShapeDtypeStruct((HNB, 1), jnp.int32),
        ),
        scratch_shapes=[pltpu.VMEM((HBS, _LANES), jnp.float32)],
        compiler_params=pltpu.CompilerParams(
            vmem_limit_bytes=64 * 1024 * 1024),
    )(flat_h, wr_pad)


def _sc_scatter_rows(flat_h, idx):
    bpw = HNA // NW

    @functools.partial(
        pl.kernel,
        mesh=plsc.VectorSubcoreMesh(core_axis_name="c", subcore_axis_name="s"),
        out_type=jax.ShapeDtypeStruct((HPAD, D), jnp.float32),
        scratch_types=[
            pltpu.VMEM((SC_CH,), jnp.int32),
            pltpu.VMEM((SC_CH, D), jnp.float32),
            pltpu.SemaphoreType.DMA,
        ],
    )
    def k(x_hbm, idx_hbm, xs_hbm, idx_v, rows_v, sem):
        wid = lax.axis_index("s") * 2 + lax.axis_index("c")
        base = wid * bpw

        @pl.loop(0, bpw // SC_CH)
        def _(ci):
            off = base + ci * SC_CH
            xoff = lax.rem(off, HBS)
            pltpu.sync_copy(idx_hbm.at[pl.ds(off, SC_CH)], idx_v)
            pltpu.sync_copy(x_hbm.at[pl.ds(xoff, SC_CH)], rows_v)
            pltpu.async_copy(rows_v, xs_hbm.at[idx_v], sem).wait()

    return k(flat_h, idx)


def _grouped_body(be_ref, xs_ref, wg_ref, wu_ref, wd_ref, ys_ref):
    xb = xs_ref[...]
    g = jnp.dot(xb, wg_ref[0], preferred_element_type=jnp.float32)
    u = jnp.dot(xb, wu_ref[0], preferred_element_type=jnp.float32)
    h = g * jax.nn.sigmoid(g) * u
    ys_ref[...] = jnp.dot(h, wd_ref[0], preferred_element_type=jnp.float32)


def _grouped_mlp(be, xs, wg, wu, wd):
    grid_spec = pltpu.PrefetchScalarGridSpec(
        num_scalar_prefetch=1,
        grid=(HNB,),
        in_specs=[
            pl.BlockSpec((G, D), lambda i, be: (i, 0)),
            pl.BlockSpec((1, D, F), lambda i, be: (be[i], 0, 0)),
            pl.BlockSpec((1, D, F), lambda i, be: (be[i], 0, 0)),
            pl.BlockSpec((1, F, D), lambda i, be: (be[i], 0, 0)),
        ],
        out_specs=pl.BlockSpec((G, D), lambda i, be: (i, 0)),
    )
    return pl.pallas_call(
        _grouped_body,
        grid_spec=grid_spec,
        out_shape=jax.ShapeDtypeStruct((HPAD, D), jnp.float32),
        compiler_params=pltpu.CompilerParams(
            dimension_semantics=("arbitrary",)),
    )(be, xs, wg, wu, wd)


def _sc_gather_rows(ys, idx):
    bpw = HNA // NW

    @functools.partial(
        pl.kernel,
        mesh=plsc.VectorSubcoreMesh(core_axis_name="c", subcore_axis_name="s"),
        out_type=jax.ShapeDtypeStruct((HNA, D), jnp.float32),
        scratch_types=[
            pltpu.VMEM((SC_CH,), jnp.int32),
            pltpu.VMEM((SC_CH, D), jnp.float32),
            pltpu.SemaphoreType.DMA,
        ],
    )
    def k(ys_hbm, idx_hbm, g_hbm, idx_v, rows_v, sem):
        wid = lax.axis_index("s") * 2 + lax.axis_index("c")
        base = wid * bpw

        @pl.loop(0, bpw // SC_CH)
        def _(ci):
            off = base + ci * SC_CH
            pltpu.sync_copy(idx_hbm.at[pl.ds(off, SC_CH)], idx_v)
            pltpu.async_copy(ys_hbm.at[idx_v], rows_v, sem).wait()
            pltpu.sync_copy(rows_v, g_hbm.at[pl.ds(off, SC_CH)])

    return k(ys, idx)


def _combine_body(g0_ref, g1_ref, w0_ref, w1_ref, out_ref):
    out_ref[...] = w0_ref[...] * g0_ref[...] + w1_ref[...] * g1_ref[...]


_RB = 512


def _combine(g, w0, w1):
    nblk = HBS // _RB
    return pl.pallas_call(
        _combine_body,
        grid=(nblk,),
        in_specs=[
            pl.BlockSpec((_RB, D), lambda i: (i, 0)),
            pl.BlockSpec((_RB, D), lambda i: (i + nblk, 0)),
            pl.BlockSpec((_RB, 1), lambda i: (i, 0)),
            pl.BlockSpec((_RB, 1), lambda i: (i, 0)),
        ],
        out_specs=pl.BlockSpec((_RB, D), lambda i: (i, 0)),
        out_shape=jax.ShapeDtypeStruct((HBS, D), jnp.float32),
    )(g, g, w0, w1)


def kernel(hidden_states, W_router, W_gate, W_up, W_down):
    B, S, H = hidden_states.shape
    flat = hidden_states.reshape(BS, D)
    wr_pad = jnp.pad(W_router, ((0, 0), (0, _LANES - NE)))

    plans = []
    for h in range(2):
        flat_h = lax.slice(flat, (h * HBS, 0), ((h + 1) * HBS, D))
        idx, w0, w1, be = _route_plan(flat_h, wr_pad)
        plans.append((flat_h, idx.reshape(HNA), w0, w1, be.reshape(HNB)))

    halves = []
    for h in range(2):
        flat_h, idx, w0, w1, be = plans[h]
        xs = _sc_scatter_rows(flat_h, idx)
        ys = _grouped_mlp(be, xs, W_gate, W_up, W_down)
        g = _sc_gather_rows(ys, idx)
        halves.append(_combine(g, w0, w1))

    out = jnp.concatenate(halves, axis=0)
    return out.reshape(B, S, H)


# speedup vs baseline: 1.2757x; 1.2675x over previous
import functools

import jax
import jax.numpy as jnp
from jax import lax
from jax.experimental import pallas as pl
from jax.experimental.pallas import tpu as pltpu
from jax.experimental.pallas import tpu_sc as plsc

NE = 8
D = 1024
F = 1024
BS = 4096
NA = 2 * BS
G = 256
NB = (NA + NE * (G - 1) + G - 1) // G
PAD = NB * G

_LANES = 128

NW = 32
SC_CH = 32
NCH = NA // NW // SC_CH


def _route_body(x_ref, wr_ref, idx_ref, w0_ref, w1_ref, be_ref, rank_ref):
    x = x_ref[...]
    wr = wr_ref[...]
    logits = jnp.dot(x, wr, preferred_element_type=jnp.float32)
    lane = lax.broadcasted_iota(jnp.int32, (BS, _LANES), 1)
    valid = lane < NE
    neg = jnp.float32(-1e30)
    ml = jnp.where(valid, logits, neg)

    m1 = jnp.max(ml, axis=1, keepdims=True)
    idx1 = jnp.min(jnp.where(ml == m1, lane, _LANES), axis=1, keepdims=True)
    oh0 = (lane == idx1).astype(jnp.float32)
    ml2 = jnp.where(lane == idx1, neg, ml)
    m2 = jnp.max(ml2, axis=1, keepdims=True)
    idx2 = jnp.min(jnp.where(ml2 == m2, lane, _LANES), axis=1, keepdims=True)
    oh1 = (lane == idx2).astype(jnp.float32)

    w0_ref[...] = jax.nn.sigmoid(m1 - m2)
    w1_ref[...] = jax.nn.sigmoid(m2 - m1)

    rank_ref[...] = oh0 + oh1
    tri = (lax.broadcasted_iota(jnp.int32, (128, 128), 0) >
           lax.broadcasted_iota(jnp.int32, (128, 128), 1)).astype(jnp.float32)

    def chunk(c, carry):
        ch = rank_ref[pl.ds(c * 128, 128), :]
        within = jnp.dot(tri, ch, preferred_element_type=jnp.float32)
        rank_ref[pl.ds(c * 128, 128), :] = within + carry
        return carry + jnp.sum(ch, axis=0, keepdims=True)

    counts = lax.fori_loop(0, BS // 128, chunk,
                           jnp.zeros((1, _LANES), jnp.float32))
    rank = rank_ref[...]

    pc = jnp.floor((counts + (G - 1)) * (1.0 / G)) * G
    upper = (lax.broadcasted_iota(jnp.int32, (128, 128), 0) <
             lax.broadcasted_iota(jnp.int32, (128, 128), 1)).astype(jnp.float32)
    seg = jnp.dot(pc, upper, preferred_element_type=jnp.float32)

    pos = seg + rank
    idx_ref[0:BS, :] = jnp.sum(pos * oh0, axis=1,
                               keepdims=True).astype(jnp.int32)
    idx_ref[BS:NA, :] = jnp.sum(pos * oh1, axis=1,
                                keepdims=True).astype(jnp.int32)

    bstart = (lax.broadcasted_iota(jnp.int32, (NB, _LANES), 0) * G
              ).astype(jnp.float32)
    lane_b = lax.broadcasted_iota(jnp.int32, (NB, _LANES), 1)
    ind = (bstart >= seg) & (bstart < seg + pc)
    be_ref[...] = jnp.sum(
        jnp.where(ind, lane_b, 0), axis=1, keepdims=True).astype(jnp.int32)


def _route_plan(flat, wr_pad):
    return pl.pallas_call(
        _route_body,
        out_shape=(
            jax.ShapeDtypeStruct((NA, 1), jnp.int32),
            jax.ShapeDtypeStruct((BS, 1), jnp.float32),
            jax.ShapeDtypeStruct((BS, 1), jnp.float32),
            jax.ShapeDtypeStruct((NB, 1), jnp.int32),
        ),
        scratch_shapes=[pltpu.VMEM((BS, _LANES), jnp.float32)],
        compiler_params=pltpu.CompilerParams(
            vmem_limit_bytes=64 * 1024 * 1024),
    )(flat, wr_pad)


def _sc_scatter_rows(flat, idx):
    bpw = NA // NW

    @functools.partial(
        pl.kernel,
        mesh=plsc.VectorSubcoreMesh(core_axis_name="c", subcore_axis_name="s"),
        out_type=jax.ShapeDtypeStruct((PAD, D), jnp.float32),
        scratch_types=[
            pltpu.VMEM((2, SC_CH), jnp.int32),
            pltpu.VMEM((2, SC_CH, D), jnp.float32),
            pltpu.SemaphoreType.DMA,
            pltpu.SemaphoreType.DMA,
            pltpu.SemaphoreType.DMA,
            pltpu.SemaphoreType.DMA,
            pltpu.SemaphoreType.DMA,
            pltpu.SemaphoreType.DMA,
        ],
    )
    def k(x_hbm, idx_hbm, xs_hbm, idx_v, rows_v, i0, i1, x0, x1, s0, s1):
        wid = lax.axis_index("s") * 2 + lax.axis_index("c")
        base = wid * bpw
        isems, xsems, ssems = (i0, i1), (x0, x1), (s0, s1)

        def start_in(ci):
            b = ci % 2
            off = base + ci * SC_CH
            xoff = lax.rem(off, BS)
            a = pltpu.make_async_copy(idx_hbm.at[pl.ds(off, SC_CH)],
                                      idx_v.at[b], isems[b])
            a.start()
            c = pltpu.make_async_copy(x_hbm.at[pl.ds(xoff, SC_CH)],
                                      rows_v.at[b], xsems[b])
            c.start()
            return a, c

        ins = {0: start_in(0)}
        scats = {}
        for ci in range(NCH):
            b = ci % 2
            a, c = ins.pop(ci)
            a.wait()
            c.wait()
            if ci >= 1:
                scats.pop(ci - 1).wait()
            if ci + 1 < NCH:
                ins[ci + 1] = start_in(ci + 1)
            s = pltpu.make_async_copy(rows_v.at[b], xs_hbm.at[idx_v.at[b]],
                                      ssems[b])
            s.start()
            scats[ci] = s
        scats.pop(NCH - 1).wait()

    return k(flat, idx)


def _grouped_body(be_ref, xs_ref, wg_ref, wu_ref, wd_ref, ys_ref):
    xb = xs_ref[...]
    g = jnp.dot(xb, wg_ref[0], preferred_element_type=jnp.float32)
    u = jnp.dot(xb, wu_ref[0], preferred_element_type=jnp.float32)
    h = g * jax.nn.sigmoid(g) * u
    ys_ref[...] = jnp.dot(h, wd_ref[0], preferred_element_type=jnp.float32)


def _grouped_mlp(be, xs, wg, wu, wd):
    grid_spec = pltpu.PrefetchScalarGridSpec(
        num_scalar_prefetch=1,
        grid=(NB,),
        in_specs=[
            pl.BlockSpec((G, D), lambda i, be: (i, 0)),
            pl.BlockSpec((1, D, F), lambda i, be: (be[i], 0, 0)),
            pl.BlockSpec((1, D, F), lambda i, be: (be[i], 0, 0)),
            pl.BlockSpec((1, F, D), lambda i, be: (be[i], 0, 0)),
        ],
        out_specs=pl.BlockSpec((G, D), lambda i, be: (i, 0)),
    )
    return pl.pallas_call(
        _grouped_body,
        grid_spec=grid_spec,
        out_shape=jax.ShapeDtypeStruct((PAD, D), jnp.float32),
        compiler_params=pltpu.CompilerParams(
            dimension_semantics=("arbitrary",)),
    )(be, xs, wg, wu, wd)


def _sc_gather_rows(ys, idx):
    bpw = NA // NW

    @functools.partial(
        pl.kernel,
        mesh=plsc.VectorSubcoreMesh(core_axis_name="c", subcore_axis_name="s"),
        out_type=jax.ShapeDtypeStruct((NA, D), jnp.float32),
        scratch_types=[
            pltpu.VMEM((2, SC_CH), jnp.int32),
            pltpu.VMEM((2, SC_CH, D), jnp.float32),
            pltpu.SemaphoreType.DMA,
            pltpu.SemaphoreType.DMA,
            pltpu.SemaphoreType.DMA,
            pltpu.SemaphoreType.DMA,
            pltpu.SemaphoreType.DMA,
            pltpu.SemaphoreType.DMA,
        ],
    )
    def k(ys_hbm, idx_hbm, g_hbm, idx_v, rows_v, i0, i1, g0, g1, o0, o1):
        wid = lax.axis_index("s") * 2 + lax.axis_index("c")
        base = wid * bpw
        isems, gsems, osems = (i0, i1), (g0, g1), (o0, o1)

        def start_idx(ci):
            b = ci % 2
            off = base + ci * SC_CH
            a = pltpu.make_async_copy(idx_hbm.at[pl.ds(off, SC_CH)],
                                      idx_v.at[b], isems[b])
            a.start()
            return a

        ins = {0: start_idx(0)}
        gats = {}
        outs = {}
        for ci in range(NCH):
            b = ci % 2
            ins.pop(ci).wait()
            if ci >= 1:
                gats.pop(ci - 1).wait()
                o = pltpu.make_async_copy(
                    rows_v.at[1 - b],
                    g_hbm.at[pl.ds(base + (ci - 1) * SC_CH, SC_CH)],
                    osems[1 - b])
                o.start()
                outs[ci - 1] = o
            if ci + 1 < NCH:
                ins[ci + 1] = start_idx(ci + 1)
            if ci >= 2:
                outs.pop(ci - 2).wait()
            gat = pltpu.make_async_copy(ys_hbm.at[idx_v.at[b]], rows_v.at[b],
                                        gsems[b])
            gat.start()
            gats[ci] = gat
        b = (NCH - 1) % 2
        gats.pop(NCH - 1).wait()
        if NCH >= 2:
            outs.pop(NCH - 2).wait()
        o = pltpu.make_async_copy(
            rows_v.at[b], g_hbm.at[pl.ds(base + (NCH - 1) * SC_CH, SC_CH)],
            osems[b])
        o.start()
        o.wait()

    return k(ys, idx)


def _combine_body(g0_ref, g1_ref, w0_ref, w1_ref, out_ref):
    out_ref[...] = w0_ref[...] * g0_ref[...] + w1_ref[...] * g1_ref[...]


_RB = 512


def _combine(g, w0, w1):
    nblk = BS // _RB
    return pl.pallas_call(
        _combine_body,
        grid=(nblk,),
        in_specs=[
            pl.BlockSpec((_RB, D), lambda i: (i, 0)),
            pl.BlockSpec((_RB, D), lambda i: (i + nblk, 0)),
            pl.BlockSpec((_RB, 1), lambda i: (i, 0)),
            pl.BlockSpec((_RB, 1), lambda i: (i, 0)),
        ],
        out_specs=pl.BlockSpec((_RB, D), lambda i: (i, 0)),
        out_shape=jax.ShapeDtypeStruct((BS, D), jnp.float32),
    )(g, g, w0, w1)


def kernel(hidden_states, W_router, W_gate, W_up, W_down):
    B, S, H = hidden_states.shape
    flat = hidden_states.reshape(BS, D)
    wr_pad = jnp.pad(W_router, ((0, 0), (0, _LANES - NE)))

    idx, w0, w1, be = _route_plan(flat, wr_pad)
    idx = idx.reshape(NA)

    xs = _sc_scatter_rows(flat, idx)
    ys = _grouped_mlp(be.reshape(NB), xs, W_gate, W_up, W_down)
    g = _sc_gather_rows(ys, idx)
    out = _combine(g, w0, w1)
    return out.reshape(B, S, H)
